# baseline (speedup 1.0000x reference)
import functools

import jax
import jax.numpy as jnp
from jax import lax
from jax.experimental import pallas as pl
from jax.experimental.pallas import tpu as pltpu
from jax.experimental.pallas import tpu_sc as plsc

_BACK = 1024
_FORW = 7168

_NC = 2
_NS = 16
_NW = _NC * _NS
_L = 16

_B = 4
_N = 8192
_D = 1024
_OUT_ROWS = _FORW - _BACK
_TOT_ROWS = _B * _OUT_ROWS

_SC_ROWS = 3072
_RPW = _SC_ROWS // _NW
_C = 32
_NCHUNK = _RPW // _C
_SC_IN0 = (_B - 1) * _N + _FORW - _SC_ROWS
_SC_BASE = (_B - 1) * _N

_TC_ROWS = _TOT_ROWS - _SC_ROWS
_BR = 1024
_TC_NBLK = _TC_ROWS // _BR


def _sc_body(x_hbm, o_hbm, base_v, b0, b1, b2, si0, si1, si2, so0, so1, so2):
    wid = lax.axis_index("s") * _NC + lax.axis_index("c")
    in_row0 = _SC_IN0 + wid * _RPW
    out_row0 = wid * _RPW

    bufs = (b0, b1, b2)
    isems = (si0, si1, si2)
    osems = (so0, so1, so2)

    pltpu.sync_copy(x_hbm.at[_SC_BASE, :], base_v)

    def start_in(i, g):
        pltpu.async_copy(x_hbm.at[pl.ds(in_row0 + g * _C, _C), :], bufs[i], isems[i])

    def wait_in(i, g):
        pltpu.make_async_copy(
            x_hbm.at[pl.ds(in_row0 + g * _C, _C), :], bufs[i], isems[i]
        ).wait()

    def start_out(i, g):
        pltpu.async_copy(bufs[i], o_hbm.at[pl.ds(out_row0 + g * _C, _C), :], osems[i])

    def wait_out(i, g):
        pltpu.make_async_copy(
            bufs[i], o_hbm.at[pl.ds(out_row0 + g * _C, _C), :], osems[i]
        ).wait()

    def compute(i):
        buf = bufs[i]

        def col_body(c, carry):
            sl = pl.ds(c * _L, _L)
            bvec = base_v[sl]

            @plsc.parallel_loop(0, _C, unroll=8)
            def _(r):
                buf[r, sl] = buf[r, sl] - bvec

            return carry

        lax.fori_loop(0, _D // _L, col_body, 0)

    start_in(0, 0)
    start_in(1, 1)
    wait_in(0, 0)
    compute(0)
    start_out(0, 0)
    start_in(2, 2)

    def loop_body(t, carry):
        for s in range(3):
            g = 3 * t + s + 1
            i = (s + 1) % 3
            jbuf = (i + 2) % 3
            wait_in(i, g)
            compute(i)
            start_out(i, g)
            wait_out(jbuf, g - 1)
            start_in(jbuf, g + 2)
        return carry

    lax.fori_loop(0, (_NCHUNK - 3) // 3, loop_body, 0)

    wait_in(1, _NCHUNK - 2)
    compute(1)
    start_out(1, _NCHUNK - 2)
    wait_out(0, _NCHUNK - 3)
    wait_in(2, _NCHUNK - 1)
    compute(2)
    start_out(2, _NCHUNK - 1)
    wait_out(1, _NCHUNK - 2)
    wait_out(2, _NCHUNK - 1)


def _sc_part(xr):
    k = functools.partial(
        pl.kernel,
        out_type=jax.ShapeDtypeStruct((_SC_ROWS, _D), jnp.float32),
        mesh=plsc.VectorSubcoreMesh(core_axis_name="c", subcore_axis_name="s"),
        scratch_types=[
            pltpu.VMEM((_D,), jnp.float32),
            pltpu.VMEM((_C, _D), jnp.float32),
            pltpu.VMEM((_C, _D), jnp.float32),
            pltpu.VMEM((_C, _D), jnp.float32),
            pltpu.SemaphoreType.DMA,
            pltpu.SemaphoreType.DMA,
            pltpu.SemaphoreType.DMA,
            pltpu.SemaphoreType.DMA,
            pltpu.SemaphoreType.DMA,
            pltpu.SemaphoreType.DMA,
        ],
    )(_sc_body)
    return k(xr)


def _tc_body(base_ref, x_ref, o_ref):
    t = pl.program_id(0)
    b = t // (_OUT_ROWS // _BR)
    o_ref[...] = x_ref[...] - base_ref[pl.ds(b, 1), :]


def _tc_part(xr, base):
    return pl.pallas_call(
        _tc_body,
        grid=(_TC_NBLK,),
        in_specs=[
            pl.BlockSpec(memory_space=pltpu.VMEM),
            pl.BlockSpec(
                (_BR, _D),
                lambda t: ((_N // _BR) * (t // (_OUT_ROWS // _BR))
                           + (_BACK // _BR) + (t % (_OUT_ROWS // _BR)), 0),
            ),
        ],
        out_specs=pl.BlockSpec((_BR, _D), lambda t: (t, 0)),
        out_shape=jax.ShapeDtypeStruct((_TOT_ROWS, _D), jnp.float32),
    )(base, xr)


_CR = 512
_RING = 8
_PF = 6
_CPB = _OUT_ROWS // _CR
_NCH = _B * _CPB


def _tcm_body(base_ref, x_hbm, o_hbm, *rest):
    bufs = rest[:_RING]
    isems = rest[_RING:2 * _RING]
    osems = rest[2 * _RING:3 * _RING]

    def in_row(g):
        return (g // _CPB) * _N + _BACK + (g % _CPB) * _CR

    def start_in(i, g):
        pltpu.async_copy(x_hbm.at[pl.ds(in_row(g), _CR), :], bufs[i], isems[i])

    def wait_in(i, g):
        pltpu.make_async_copy(
            x_hbm.at[pl.ds(in_row(g), _CR), :], bufs[i], isems[i]
        ).wait()

    def start_out(i, g):
        pltpu.async_copy(bufs[i], o_hbm.at[pl.ds(g * _CR, _CR), :], osems[i])

    def wait_out(i, g):
        pltpu.make_async_copy(
            bufs[i], o_hbm.at[pl.ds(g * _CR, _CR), :], osems[i]
        ).wait()

    def compute(i, g):
        bvec = base_ref[pl.ds(g // _CPB, 1), :]
        bufs[i][...] = bufs[i][...] - bvec

    for i in range(_PF):
        start_in(i, i)

    def loop_body(t, carry):
        for s in range(_RING):
            g = _RING * t + s
            wait_in(s, g)
            compute(s, g)
            start_out(s, g)
            nxt = (s + _PF) % _RING

            @pl.when(g + _PF < _NCH)
            def _():
                @pl.when(g >= _RING - _PF)
                def _():
                    wait_out(nxt, g + _PF - _RING)

                start_in(nxt, g + _PF)

        return carry

    lax.fori_loop(0, _NCH // _RING, loop_body, 0)

    for i in range(_RING):
        g = _NCH - _RING + i
        wait_out(g % _RING, g)


def _tcm_kernel(x):
    xr = x.reshape(_B * _N, _D)
    base = x[:, 0, :]
    out = pl.pallas_call(
        _tcm_body,
        in_specs=[
            pl.BlockSpec(memory_space=pltpu.VMEM),
            pl.BlockSpec(memory_space=pl.ANY),
        ],
        out_specs=pl.BlockSpec(memory_space=pl.ANY),
        out_shape=jax.ShapeDtypeStruct((_TOT_ROWS, _D), jnp.float32),
        scratch_shapes=(
            [pltpu.VMEM((_CR, _D), jnp.float32)] * _RING
            + [pltpu.SemaphoreType.DMA] * (2 * _RING)
        ),
    )(base, xr)
    return out.reshape(_B, _OUT_ROWS, _D)


def kernel(x):
    return _tcm_kernel(x)


# speedup vs baseline: 1.0031x; 1.0031x over previous
import functools

import jax
import jax.numpy as jnp
from jax import lax
from jax.experimental import pallas as pl
from jax.experimental.pallas import tpu as pltpu
from jax.experimental.pallas import tpu_sc as plsc

_BACK = 1024
_FORW = 7168

_NC = 2
_NS = 16
_NW = _NC * _NS
_L = 16

_B = 4
_N = 8192
_D = 1024
_OUT_ROWS = _FORW - _BACK
_TOT_ROWS = _B * _OUT_ROWS

_SC_ROWS = 3072
_RPW = _SC_ROWS // _NW
_C = 32
_NCHUNK = _RPW // _C
_SC_IN0 = (_B - 1) * _N + _FORW - _SC_ROWS
_SC_BASE = (_B - 1) * _N

_TC_ROWS = _TOT_ROWS - _SC_ROWS
_BR = 1024
_TC_NBLK = _TC_ROWS // _BR


def _sc_body(x_hbm, o_hbm, base_v, b0, b1, b2, si0, si1, si2, so0, so1, so2):
    wid = lax.axis_index("s") * _NC + lax.axis_index("c")
    in_row0 = _SC_IN0 + wid * _RPW
    out_row0 = wid * _RPW

    bufs = (b0, b1, b2)
    isems = (si0, si1, si2)
    osems = (so0, so1, so2)

    pltpu.sync_copy(x_hbm.at[_SC_BASE, :], base_v)

    def start_in(i, g):
        pltpu.async_copy(x_hbm.at[pl.ds(in_row0 + g * _C, _C), :], bufs[i], isems[i])

    def wait_in(i, g):
        pltpu.make_async_copy(
            x_hbm.at[pl.ds(in_row0 + g * _C, _C), :], bufs[i], isems[i]
        ).wait()

    def start_out(i, g):
        pltpu.async_copy(bufs[i], o_hbm.at[pl.ds(out_row0 + g * _C, _C), :], osems[i])

    def wait_out(i, g):
        pltpu.make_async_copy(
            bufs[i], o_hbm.at[pl.ds(out_row0 + g * _C, _C), :], osems[i]
        ).wait()

    def compute(i):
        buf = bufs[i]

        def col_body(c, carry):
            sl = pl.ds(c * _L, _L)
            bvec = base_v[sl]

            @plsc.parallel_loop(0, _C, unroll=8)
            def _(r):
                buf[r, sl] = buf[r, sl] - bvec

            return carry

        lax.fori_loop(0, _D // _L, col_body, 0)

    start_in(0, 0)
    start_in(1, 1)
    wait_in(0, 0)
    compute(0)
    start_out(0, 0)
    start_in(2, 2)

    def loop_body(t, carry):
        for s in range(3):
            g = 3 * t + s + 1
            i = (s + 1) % 3
            jbuf = (i + 2) % 3
            wait_in(i, g)
            compute(i)
            start_out(i, g)
            wait_out(jbuf, g - 1)
            start_in(jbuf, g + 2)
        return carry

    lax.fori_loop(0, (_NCHUNK - 3) // 3, loop_body, 0)

    wait_in(1, _NCHUNK - 2)
    compute(1)
    start_out(1, _NCHUNK - 2)
    wait_out(0, _NCHUNK - 3)
    wait_in(2, _NCHUNK - 1)
    compute(2)
    start_out(2, _NCHUNK - 1)
    wait_out(1, _NCHUNK - 2)
    wait_out(2, _NCHUNK - 1)


def _sc_part(xr):
    k = functools.partial(
        pl.kernel,
        out_type=jax.ShapeDtypeStruct((_SC_ROWS, _D), jnp.float32),
        mesh=plsc.VectorSubcoreMesh(core_axis_name="c", subcore_axis_name="s"),
        scratch_types=[
            pltpu.VMEM((_D,), jnp.float32),
            pltpu.VMEM((_C, _D), jnp.float32),
            pltpu.VMEM((_C, _D), jnp.float32),
            pltpu.VMEM((_C, _D), jnp.float32),
            pltpu.SemaphoreType.DMA,
            pltpu.SemaphoreType.DMA,
            pltpu.SemaphoreType.DMA,
            pltpu.SemaphoreType.DMA,
            pltpu.SemaphoreType.DMA,
            pltpu.SemaphoreType.DMA,
        ],
    )(_sc_body)
    return k(xr)


def _tc_body(base_ref, x_ref, o_ref):
    t = pl.program_id(0)
    b = t // (_OUT_ROWS // _BR)
    o_ref[...] = x_ref[...] - base_ref[pl.ds(b, 1), :]


def _tc_part(xr, base):
    return pl.pallas_call(
        _tc_body,
        grid=(_TC_NBLK,),
        in_specs=[
            pl.BlockSpec(memory_space=pltpu.VMEM),
            pl.BlockSpec(
                (_BR, _D),
                lambda t: ((_N // _BR) * (t // (_OUT_ROWS // _BR))
                           + (_BACK // _BR) + (t % (_OUT_ROWS // _BR)), 0),
            ),
        ],
        out_specs=pl.BlockSpec((_BR, _D), lambda t: (t, 0)),
        out_shape=jax.ShapeDtypeStruct((_TOT_ROWS, _D), jnp.float32),
    )(base, xr)


_CR = 1024
_RING = 6
_PF = 4
_CPB = _OUT_ROWS // _CR
_NCH = _B * _CPB


def _tcm_body(base_ref, x_hbm, o_hbm, *rest):
    bufs = rest[:_RING]
    isems = rest[_RING:2 * _RING]
    osems = rest[2 * _RING:3 * _RING]

    def in_row(g):
        return (g // _CPB) * _N + _BACK + (g % _CPB) * _CR

    def start_in(i, g):
        pltpu.async_copy(x_hbm.at[pl.ds(in_row(g), _CR), :], bufs[i], isems[i])

    def wait_in(i, g):
        pltpu.make_async_copy(
            x_hbm.at[pl.ds(in_row(g), _CR), :], bufs[i], isems[i]
        ).wait()

    def start_out(i, g):
        pltpu.async_copy(bufs[i], o_hbm.at[pl.ds(g * _CR, _CR), :], osems[i])

    def wait_out(i, g):
        pltpu.make_async_copy(
            bufs[i], o_hbm.at[pl.ds(g * _CR, _CR), :], osems[i]
        ).wait()

    def compute(i, g):
        bvec = base_ref[pl.ds(g // _CPB, 1), :]
        bufs[i][...] = bufs[i][...] - bvec

    for i in range(_PF):
        start_in(i, i)

    def loop_body(t, carry):
        for s in range(_RING):
            g = _RING * t + s
            wait_in(s, g)
            compute(s, g)
            start_out(s, g)
            nxt = (s + _PF) % _RING

            @pl.when(g + _PF < _NCH)
            def _():
                @pl.when(g >= _RING - _PF)
                def _():
                    wait_out(nxt, g + _PF - _RING)

                start_in(nxt, g + _PF)

        return carry

    lax.fori_loop(0, _NCH // _RING, loop_body, 0)

    for i in range(_RING):
        g = _NCH - _RING + i
        wait_out(g % _RING, g)


def _tcm_kernel(x):
    xr = x.reshape(_B * _N, _D)
    base = x[:, 0, :]
    out = pl.pallas_call(
        _tcm_body,
        in_specs=[
            pl.BlockSpec(memory_space=pltpu.VMEM),
            pl.BlockSpec(memory_space=pl.ANY),
        ],
        out_specs=pl.BlockSpec(memory_space=pl.ANY),
        out_shape=jax.ShapeDtypeStruct((_TOT_ROWS, _D), jnp.float32),
        scratch_shapes=(
            [pltpu.VMEM((_CR, _D), jnp.float32)] * _RING
            + [pltpu.SemaphoreType.DMA] * (2 * _RING)
        ),
    )(base, xr)
    return out.reshape(_B, _OUT_ROWS, _D)


def kernel(x):
    return _tcm_kernel(x)
